# baseline (speedup 1.0000x reference)
import functools

import jax
import jax.numpy as jnp
import numpy as np
from jax.experimental import pallas as pl
from jax.experimental.pallas import tpu as pltpu
from jax.sharding import Mesh, PartitionSpec as P

_NF = 8
_C = 4096
_EPS = 1e-8
_T = 512
_B = _C // _T
_UPPER = (_B * (_B + 1)) // 2


def _tables(nshard):
    sizes = [(_B - x, x) for x in range(_B)]
    sizes.sort(reverse=True)
    loads = [0] * nshard
    bands = [[] for _ in range(nshard)]
    for sz, x in sizes:
        i = loads.index(min(loads))
        bands[i].append(x)
        loads[i] += sz
    steps = max(loads)
    pb_t = np.zeros((nshard, steps), np.int32)
    qb_t = np.zeros((nshard, steps), np.int32)
    first_t = np.zeros((nshard, steps), np.int32)
    last_t = np.zeros((nshard, steps), np.int32)
    for c in range(nshard):
        s = 0
        for x in bands[c]:
            for q in range(x, _B):
                pb_t[c, s] = x
                qb_t[c, s] = q
                first_t[c, s] = 1 if q == x else 0
                last_t[c, s] = 1 if q == _B - 1 else 0
                s += 1
        assert s == steps, (c, s, steps)
    return pb_t, qb_t, first_t, last_t, steps


def _acc_kernel(x_ref, xt_ref, ci_ref, pb_ref, qb_ref, fi_ref, la_ref,
                m_ref, g8_ref, g_ref, sb_ref, *, nsteps):
    s = pl.program_id(0)
    ci = ci_ref[0, 0]

    @pl.when(s == 0)
    def _init():
        m_ref[...] = jnp.zeros_like(m_ref)
        g_ref[...] = jnp.zeros_like(g_ref)

    pb = pb_ref[ci, s]
    qb = qb_ref[ci, s]
    po = pl.multiple_of(pb * _T, _T)
    qo = pl.multiple_of(qb * _T, _T)

    a = []
    for f in range(_NF):
        xrow = x_ref[f:f + 1, pl.ds(qo, _T)]
        xcol = xt_ref[pl.ds(po, _T), f:f + 1]
        a.append(jnp.abs(xcol - xrow))

    @pl.when(pb < qb)
    def _colsums():
        cs = jnp.concatenate(
            [jnp.sum(a[f], axis=0, keepdims=True) for f in range(_NF)],
            axis=0)
        m_ref[:, pl.ds(qo, _T)] += cs

    @pl.when(fi_ref[ci, s] == 1)
    def _band_start():
        for f in range(_NF):
            sb_ref[f] = a[f]

    @pl.when(fi_ref[ci, s] == 0)
    def _band_acc():
        for f in range(_NF):
            sb_ref[f] += a[f]

    @pl.when(la_ref[ci, s] == 1)
    def _band_end():
        for f in range(_NF):
            m_ref[f, pl.ds(po, _T)] += jnp.sum(sb_ref[f], axis=1)

    w = jnp.where(pb == qb, 1.0, 2.0).astype(jnp.float32)
    for f in range(_NF):
        pcs = jnp.concatenate(
            [jnp.sum(a[f] * a[g], axis=0, keepdims=True)
             for g in range(f, _NF)], axis=1)
        g_ref[f:f + 1, f * _T:] += w * pcs

    @pl.when(s == nsteps - 1)
    def _collapse():
        gfull = g_ref[...]
        g8_ref[...] = jnp.concatenate(
            [jnp.sum(gfull[:, g * _T:(g + 1) * _T], axis=1, keepdims=True)
             for g in range(_NF)], axis=1)


def _fin_kernel(m_ref, g8_ref, o_ref):
    m = m_ref[...]
    g8 = g8_ref[...]
    c2 = float(_C) * float(_C)
    ri = jax.lax.broadcasted_iota(jnp.int32, (_NF, _NF), 0)
    ci = jax.lax.broadcasted_iota(jnp.int32, (_NF, _NF), 1)
    msum = jnp.sum(m, axis=1, keepdims=True)
    m8 = jnp.concatenate(
        [jnp.sum(m * m[g:g + 1, :], axis=1, keepdims=True)
         for g in range(_NF)], axis=1)
    t = msum / c2
    trow = jnp.sum(jnp.where(ri == ci, t, 0.0), axis=0, keepdims=True)
    ttt = t * trow
    s8 = g8 * (1.0 / c2) - m8 * (2.0 / (c2 * float(_C))) + ttt
    dcov = jnp.sqrt(jnp.maximum(s8, 0.0) + _EPS)
    dmat = jnp.where(ri == ci, dcov, 0.0)
    dcol = jnp.sum(dmat, axis=1, keepdims=True)
    drow = jnp.sum(dmat, axis=0, keepdims=True)
    ratio = dcov / jnp.sqrt(dcol * drow + _EPS)
    o_ref[...] = jnp.sum(jnp.where(ci > ri, ratio, 0.0),
                         axis=(0, 1), keepdims=True)


def _acc_call(xs, xts, civ, tbls, *, nsteps):
    smem = pl.BlockSpec(memory_space=pltpu.SMEM)
    return pl.pallas_call(
        functools.partial(_acc_kernel, nsteps=nsteps),
        grid=(nsteps,),
        in_specs=[
            pl.BlockSpec((_NF, _C), lambda s: (0, 0)),
            pl.BlockSpec((_C, _NF), lambda s: (0, 0)),
            smem, smem, smem, smem, smem,
        ],
        out_specs=[
            pl.BlockSpec((_NF, _C), lambda s: (0, 0)),
            pl.BlockSpec((_NF, _NF), lambda s: (0, 0)),
        ],
        out_shape=[
            jax.ShapeDtypeStruct((_NF, _C), jnp.float32),
            jax.ShapeDtypeStruct((_NF, _NF), jnp.float32),
        ],
        scratch_shapes=[
            pltpu.VMEM((_NF, _NF * _T), jnp.float32),
            pltpu.VMEM((_NF, _T, _T), jnp.float32),
        ],
        compiler_params=pltpu.CompilerParams(
            dimension_semantics=("arbitrary",)),
    )(xs, xts, civ, *tbls)


def kernel(disen_weight_att):
    x = disen_weight_att.astype(jnp.float32)
    xt = x.T
    nshard = 2 if jax.device_count() >= 2 else 1
    mesh = Mesh(np.array(jax.devices()[:nshard]), ("c",))
    pb_t, qb_t, fi_t, la_t, nsteps = _tables(nshard)
    tbls = tuple(jnp.asarray(t) for t in (pb_t, qb_t, fi_t, la_t))

    def _shard(xs, xts):
        ci = jax.lax.axis_index("c").astype(jnp.int32)
        civ = jnp.full((1, 1), ci, jnp.int32)
        m_p, g8_p = _acc_call(xs, xts, civ, tbls, nsteps=nsteps)
        m = jax.lax.psum(m_p, "c")
        g8 = jax.lax.psum(g8_p, "c")
        return pl.pallas_call(
            _fin_kernel,
            out_shape=jax.ShapeDtypeStruct((1, 1), jnp.float32),
        )(m, g8)

    out = jax.shard_map(
        _shard, mesh=mesh,
        in_specs=(P(None, None), P(None, None)),
        out_specs=P(None, None),
        check_vma=False,
    )(x, xt)
    return out.reshape(())


# speedup vs baseline: 1.0414x; 1.0414x over previous
import functools

import jax
import jax.numpy as jnp
import numpy as np
from jax.experimental import pallas as pl
from jax.experimental.pallas import tpu as pltpu
from jax.sharding import Mesh, PartitionSpec as P

_NF = 8
_C = 4096
_EPS = 1e-8
_T = 256
_B = _C // _T
_UPPER = (_B * (_B + 1)) // 2


def _tables(nshard):
    sizes = [(_B - x, x) for x in range(_B)]
    sizes.sort(reverse=True)
    loads = [0] * nshard
    bands = [[] for _ in range(nshard)]
    for sz, x in sizes:
        i = loads.index(min(loads))
        bands[i].append(x)
        loads[i] += sz
    steps = max(loads)
    pb_t = np.zeros((nshard, steps), np.int32)
    qb_t = np.zeros((nshard, steps), np.int32)
    first_t = np.zeros((nshard, steps), np.int32)
    last_t = np.zeros((nshard, steps), np.int32)
    for c in range(nshard):
        s = 0
        for x in bands[c]:
            for q in range(x, _B):
                pb_t[c, s] = x
                qb_t[c, s] = q
                first_t[c, s] = 1 if q == x else 0
                last_t[c, s] = 1 if q == _B - 1 else 0
                s += 1
        assert s == steps, (c, s, steps)
    return pb_t, qb_t, first_t, last_t, steps


def _acc_kernel(x_ref, xt_ref, ci_ref, pb_ref, qb_ref, fi_ref, la_ref,
                m_ref, g8_ref, g_ref, sb_ref, *, nsteps):
    s = pl.program_id(0)
    ci = ci_ref[0, 0]

    @pl.when(s == 0)
    def _init():
        m_ref[...] = jnp.zeros_like(m_ref)
        g_ref[...] = jnp.zeros_like(g_ref)

    pb = pb_ref[ci, s]
    qb = qb_ref[ci, s]
    po = pl.multiple_of(pb * _T, _T)
    qo = pl.multiple_of(qb * _T, _T)

    a = []
    for f in range(_NF):
        xrow = x_ref[f:f + 1, pl.ds(qo, _T)]
        xcol = xt_ref[pl.ds(po, _T), f:f + 1]
        a.append(jnp.abs(xcol - xrow))

    @pl.when(pb < qb)
    def _colsums():
        cs = jnp.concatenate(
            [jnp.sum(a[f], axis=0, keepdims=True) for f in range(_NF)],
            axis=0)
        m_ref[:, pl.ds(qo, _T)] += cs

    @pl.when(fi_ref[ci, s] == 1)
    def _band_start():
        for f in range(_NF):
            sb_ref[f] = a[f]

    @pl.when(fi_ref[ci, s] == 0)
    def _band_acc():
        for f in range(_NF):
            sb_ref[f] += a[f]

    @pl.when(la_ref[ci, s] == 1)
    def _band_end():
        for f in range(_NF):
            m_ref[f, pl.ds(po, _T)] += jnp.sum(sb_ref[f], axis=1)

    w = jnp.where(pb == qb, 1.0, 2.0).astype(jnp.float32)
    for f in range(_NF):
        pcs = jnp.concatenate(
            [jnp.sum(a[f] * a[g], axis=0, keepdims=True)
             for g in range(f, _NF)], axis=1)
        g_ref[f:f + 1, f * _T:] += w * pcs

    @pl.when(s == nsteps - 1)
    def _collapse():
        gfull = g_ref[...]
        g8_ref[...] = jnp.concatenate(
            [jnp.sum(gfull[:, g * _T:(g + 1) * _T], axis=1, keepdims=True)
             for g in range(_NF)], axis=1)


def _fin_kernel(m_ref, g8_ref, o_ref):
    m = m_ref[...]
    g8 = g8_ref[...]
    c2 = float(_C) * float(_C)
    ri = jax.lax.broadcasted_iota(jnp.int32, (_NF, _NF), 0)
    ci = jax.lax.broadcasted_iota(jnp.int32, (_NF, _NF), 1)
    msum = jnp.sum(m, axis=1, keepdims=True)
    m8 = jnp.concatenate(
        [jnp.sum(m * m[g:g + 1, :], axis=1, keepdims=True)
         for g in range(_NF)], axis=1)
    t = msum / c2
    trow = jnp.sum(jnp.where(ri == ci, t, 0.0), axis=0, keepdims=True)
    ttt = t * trow
    s8 = g8 * (1.0 / c2) - m8 * (2.0 / (c2 * float(_C))) + ttt
    dcov = jnp.sqrt(jnp.maximum(s8, 0.0) + _EPS)
    dmat = jnp.where(ri == ci, dcov, 0.0)
    dcol = jnp.sum(dmat, axis=1, keepdims=True)
    drow = jnp.sum(dmat, axis=0, keepdims=True)
    ratio = dcov / jnp.sqrt(dcol * drow + _EPS)
    o_ref[...] = jnp.sum(jnp.where(ci > ri, ratio, 0.0),
                         axis=(0, 1), keepdims=True)


def _acc_call(xs, xts, civ, tbls, *, nsteps):
    smem = pl.BlockSpec(memory_space=pltpu.SMEM)
    return pl.pallas_call(
        functools.partial(_acc_kernel, nsteps=nsteps),
        grid=(nsteps,),
        in_specs=[
            pl.BlockSpec((_NF, _C), lambda s: (0, 0)),
            pl.BlockSpec((_C, _NF), lambda s: (0, 0)),
            smem, smem, smem, smem, smem,
        ],
        out_specs=[
            pl.BlockSpec((_NF, _C), lambda s: (0, 0)),
            pl.BlockSpec((_NF, _NF), lambda s: (0, 0)),
        ],
        out_shape=[
            jax.ShapeDtypeStruct((_NF, _C), jnp.float32),
            jax.ShapeDtypeStruct((_NF, _NF), jnp.float32),
        ],
        scratch_shapes=[
            pltpu.VMEM((_NF, _NF * _T), jnp.float32),
            pltpu.VMEM((_NF, _T, _T), jnp.float32),
        ],
        compiler_params=pltpu.CompilerParams(
            dimension_semantics=("arbitrary",)),
    )(xs, xts, civ, *tbls)


def kernel(disen_weight_att):
    x = disen_weight_att.astype(jnp.float32)
    xt = x.T
    nshard = 2 if jax.device_count() >= 2 else 1
    mesh = Mesh(np.array(jax.devices()[:nshard]), ("c",))
    pb_t, qb_t, fi_t, la_t, nsteps = _tables(nshard)
    tbls = tuple(jnp.asarray(t) for t in (pb_t, qb_t, fi_t, la_t))

    def _shard(xs, xts):
        ci = jax.lax.axis_index("c").astype(jnp.int32)
        civ = jnp.full((1, 1), ci, jnp.int32)
        m_p, g8_p = _acc_call(xs, xts, civ, tbls, nsteps=nsteps)
        m = jax.lax.psum(m_p, "c")
        g8 = jax.lax.psum(g8_p, "c")
        return pl.pallas_call(
            _fin_kernel,
            out_shape=jax.ShapeDtypeStruct((1, 1), jnp.float32),
        )(m, g8)

    out = jax.shard_map(
        _shard, mesh=mesh,
        in_specs=(P(None, None), P(None, None)),
        out_specs=P(None, None),
        check_vma=False,
    )(x, xt)
    return out.reshape(())


# speedup vs baseline: 1.0787x; 1.0359x over previous
import functools

import jax
import jax.numpy as jnp
import numpy as np
from jax.experimental import pallas as pl
from jax.experimental.pallas import tpu as pltpu
from jax.sharding import Mesh, PartitionSpec as P

_NF = 8
_C = 4096
_EPS = 1e-8
_T = 256
_B = _C // _T


def _tables(nshard):
    sizes = [(_B - x, x) for x in range(_B)]
    sizes.sort(reverse=True)
    loads = [0] * nshard
    bands = [[] for _ in range(nshard)]
    for sz, x in sizes:
        i = loads.index(min(loads))
        bands[i].append(x)
        loads[i] += sz
    steps = max(loads)
    pb_t = np.zeros((nshard, steps), np.int32)
    qb_t = np.zeros((nshard, steps), np.int32)
    first_t = np.zeros((nshard, steps), np.int32)
    last_t = np.zeros((nshard, steps), np.int32)
    for c in range(nshard):
        s = 0
        for x in bands[c]:
            for q in range(x, _B):
                pb_t[c, s] = x
                qb_t[c, s] = q
                first_t[c, s] = 1 if q == x else 0
                last_t[c, s] = 1 if q == _B - 1 else 0
                s += 1
        assert s == steps, (c, s, steps)
    return pb_t, qb_t, first_t, last_t, steps


def _acc_kernel(x_ref, xt_ref, ci_ref, pb_ref, qb_ref, fi_ref, la_ref,
                m_ref, g8_ref, g_ref, sb_ref, *, nsteps):
    s = pl.program_id(0)
    ci = ci_ref[0, 0]

    @pl.when(s == 0)
    def _init():
        m_ref[...] = jnp.zeros_like(m_ref)
        g_ref[...] = jnp.zeros_like(g_ref)

    pb = pb_ref[ci, s]
    qb = qb_ref[ci, s]
    po = pl.multiple_of(pb * _T, _T)
    qo = pl.multiple_of(qb * _T, _T)

    a = []
    for f in range(_NF):
        xrow = x_ref[f:f + 1, pl.ds(qo, _T)]
        xcol = xt_ref[pl.ds(po, _T), f:f + 1]
        a.append(jnp.abs(xcol - xrow))

    @pl.when(pb < qb)
    def _colsums():
        cs = jnp.concatenate(
            [jnp.sum(a[f], axis=0, keepdims=True) for f in range(_NF)],
            axis=0)
        m_ref[:, pl.ds(qo, _T)] += cs

    @pl.when(fi_ref[ci, s] == 1)
    def _band_start():
        for f in range(_NF):
            sb_ref[f] = a[f]

    @pl.when(fi_ref[ci, s] == 0)
    def _band_acc():
        for f in range(_NF):
            sb_ref[f] += a[f]

    @pl.when(la_ref[ci, s] == 1)
    def _band_end():
        for f in range(_NF):
            m_ref[f, pl.ds(po, _T)] += jnp.sum(sb_ref[f], axis=1)

    w = jnp.where(pb == qb, 1.0, 2.0).astype(jnp.float32)
    for f in range(_NF):
        pcs = jnp.concatenate(
            [jnp.sum(a[f] * a[g], axis=0, keepdims=True)
             for g in range(f, _NF)], axis=1)
        g_ref[f:f + 1, f * _T:] += w * pcs

    @pl.when(s == nsteps - 1)
    def _collapse():
        gfull = g_ref[...]
        g8_ref[...] = jnp.concatenate(
            [jnp.sum(gfull[:, g * _T:(g + 1) * _T], axis=1, keepdims=True)
             for g in range(_NF)], axis=1)


def _fin_kernel(m_ref, g8_ref, o_ref):
    m = m_ref[...]
    g8 = g8_ref[...]
    c2 = float(_C) * float(_C)
    ri = jax.lax.broadcasted_iota(jnp.int32, (_NF, _NF), 0)
    ci = jax.lax.broadcasted_iota(jnp.int32, (_NF, _NF), 1)
    msum = jnp.sum(m, axis=1, keepdims=True)
    m8 = jnp.concatenate(
        [jnp.sum(m * m[g:g + 1, :], axis=1, keepdims=True)
         for g in range(_NF)], axis=1)
    t = msum / c2
    trow = jnp.sum(jnp.where(ri == ci, t, 0.0), axis=0, keepdims=True)
    ttt = t * trow
    s8 = g8 * (1.0 / c2) - m8 * (2.0 / (c2 * float(_C))) + ttt
    dcov = jnp.sqrt(jnp.maximum(s8, 0.0) + _EPS)
    dmat = jnp.where(ri == ci, dcov, 0.0)
    dcol = jnp.sum(dmat, axis=1, keepdims=True)
    drow = jnp.sum(dmat, axis=0, keepdims=True)
    ratio = dcov / jnp.sqrt(dcol * drow + _EPS)
    o_ref[...] = jnp.sum(jnp.where(ci > ri, ratio, 0.0),
                         axis=(0, 1), keepdims=True)


def _acc_call(xs, xts, civ, tbls, *, nsteps):
    smem = pl.BlockSpec(memory_space=pltpu.SMEM)
    return pl.pallas_call(
        functools.partial(_acc_kernel, nsteps=nsteps),
        grid=(nsteps,),
        in_specs=[
            pl.BlockSpec((_NF, _C), lambda s: (0, 0)),
            pl.BlockSpec((_C, _NF), lambda s: (0, 0)),
            smem, smem, smem, smem, smem,
        ],
        out_specs=[
            pl.BlockSpec((_NF, _C), lambda s: (0, 0)),
            pl.BlockSpec((_NF, _NF), lambda s: (0, 0)),
        ],
        out_shape=[
            jax.ShapeDtypeStruct((_NF, _C), jnp.float32),
            jax.ShapeDtypeStruct((_NF, _NF), jnp.float32),
        ],
        scratch_shapes=[
            pltpu.VMEM((_NF, _NF * _T), jnp.float32),
            pltpu.VMEM((_NF, _T, _T), jnp.float32),
        ],
        compiler_params=pltpu.CompilerParams(
            dimension_semantics=("arbitrary",)),
    )(xs, xts, civ, *tbls)


def kernel(disen_weight_att):
    x = disen_weight_att.astype(jnp.float32)
    xt = x.T
    nshard = 2 if jax.device_count() >= 2 else 1
    mesh = Mesh(np.array(jax.devices()[:nshard]), ("c",))
    pb_t, qb_t, fi_t, la_t, nsteps = _tables(nshard)
    tbls = tuple(jnp.asarray(t) for t in (pb_t, qb_t, fi_t, la_t))

    def _shard(xs, xts):
        ci = jax.lax.axis_index("c").astype(jnp.int32)
        civ = jnp.full((1, 1), ci, jnp.int32)
        m_p, g8_p = _acc_call(xs, xts, civ, tbls, nsteps=nsteps)
        m = jax.lax.psum(m_p, "c")
        g8 = jax.lax.psum(g8_p, "c")
        return pl.pallas_call(
            _fin_kernel,
            out_shape=jax.ShapeDtypeStruct((1, 1), jnp.float32),
        )(m, g8)

    out = jax.shard_map(
        _shard, mesh=mesh,
        in_specs=(P(None, None), P(None, None)),
        out_specs=P(None, None),
        check_vma=False,
    )(x, xt)
    return out.reshape(())
